# baseline (speedup 1.0000x reference)
import functools

import jax
import jax.numpy as jnp
from jax import lax
from jax.experimental import pallas as pl
from jax.experimental.pallas import tpu as pltpu
from jax.experimental.pallas import tpu_sc as plsc

NUM_SEGMENTS = 512
NC = 2
NS = 16
L = 16
CR = 32

_NEG_INF = float("-inf")


@functools.lru_cache(maxsize=None)
def _make_seg_reduce(n, d, b):
    nw = NC * NS
    seg_per_w = b // nw
    n_sub = d // L
    assert n % 8 == 0 and n >= CR and d % L == 0 and b % nw == 0
    mesh = plsc.VectorSubcoreMesh(core_axis_name="c", subcore_axis_name="s",
                                  num_cores=NC, num_subcores=NS)

    @functools.partial(
        pl.kernel,
        out_type=(
            jax.ShapeDtypeStruct((b, d), jnp.float32),
            jax.ShapeDtypeStruct((b, d), jnp.float32),
        ),
        mesh=mesh,
        scratch_types=[
            pltpu.VMEM((n,), jnp.int32),
            pltpu.VMEM((8, d), jnp.float32),
            pltpu.VMEM((CR, d), jnp.float32),
            pltpu.VMEM((CR, d), jnp.float32),
            pltpu.VMEM((CR, d), jnp.float32),
            pltpu.VMEM((CR, d), jnp.float32),
            pltpu.VMEM((CR, d), jnp.float32),
            pltpu.VMEM((seg_per_w, d), jnp.float32),
            pltpu.VMEM((seg_per_w, d), jnp.float32),
            pltpu.SemaphoreType.DMA,
            pltpu.SemaphoreType.DMA,
            pltpu.SemaphoreType.DMA,
            pltpu.SemaphoreType.DMA,
            pltpu.SemaphoreType.DMA,
            pltpu.SemaphoreType.DMA,
        ],
        compiler_params=pltpu.CompilerParams(needs_layout_passes=False),
    )
    def seg_reduce(memb_hbm, x_hbm, max_hbm, sum_hbm,
                   memb_v, hbuf, fb0, fb1, fb2, fb3, tbuf, omax_v, osum_v,
                   sem_h, sem_f0, sem_f1, sem_f2, sem_f3, sem_t):
        wid = lax.axis_index("c") * NS + lax.axis_index("s")
        seg0 = wid * seg_per_w

        pltpu.sync_copy(memb_hbm, memb_v)

        targets = seg0 + lax.iota(jnp.int32, L)

        def lower_bound(tv):
            def step(_, carry):
                lo, hi = carry
                mid = lax.div(lo + hi, 2)
                vals = plsc.load_gather(memb_v, [jnp.minimum(mid, n - 1)])
                pred = vals < tv
                return jnp.where(pred, mid + 1, lo), jnp.where(pred, hi, mid)
            lo = jnp.zeros((L,), jnp.int32)
            hi = jnp.full((L,), n, jnp.int32)
            lo, hi = lax.fori_loop(0, 17, step, (lo, hi))
            return jnp.minimum(lo, n)

        starts = lower_bound(targets)
        ends = lower_bound(targets + 1)
        lanes = lax.iota(jnp.int32, L)

        def lane_extract(vec, idx):
            return jnp.sum(jnp.where(lanes == idx, vec, 0), axis=0)

        def accum_rows(s_idx, buf, nrows, valid_of_row):
            npart = 4 if nrows >= 8 else 2
            for c in range(n_sub):
                cs = pl.ds(c * L, L)
                pmax = [jnp.full((L,), _NEG_INF, jnp.float32)
                        for _ in range(npart)]
                psum = [jnp.zeros((L,), jnp.float32) for _ in range(npart)]
                for r in range(nrows):
                    v = buf[r, cs]
                    vmask = valid_of_row(r)
                    if vmask is None:
                        vm = vs = v
                    else:
                        vm = jnp.where(vmask, v, _NEG_INF)
                        vs = jnp.where(vmask, v, 0.0)
                    p = r % npart
                    pmax[p] = jnp.maximum(pmax[p], vm)
                    psum[p] = psum[p] + vs
                for p in range(1, npart):
                    pmax[0] = jnp.maximum(pmax[0], pmax[p])
                    psum[0] = psum[0] + psum[p]
                omax_v[s_idx, cs] = jnp.maximum(omax_v[s_idx, cs], pmax[0])
                osum_v[s_idx, cs] = osum_v[s_idx, cs] + psum[0]

        def do_segment(s_idx, _):
            lo = lane_extract(starts, s_idx)
            hi = lane_extract(ends, s_idx)
            lo8u = lax.div(lo + 7, 8) * 8
            nfull = lax.div(jnp.maximum(hi - lo8u, 0), CR)
            head_hi = jnp.minimum(lo8u, hi)
            head_needed = lo < head_hi
            hbase = jnp.minimum(jnp.maximum(lo8u - 8, 0), n - 8)
            t_lo = lo8u + nfull * CR
            tail_needed = t_lo < hi
            tbase = jnp.minimum(t_lo, n - CR)

            def start_chunk(k, buf, sem):
                base = pl.multiple_of(lo8u + k * CR, 8)
                pltpu.make_async_copy(
                    x_hbm.at[pl.ds(base, CR)], buf, sem).start()

            fbufs = [fb0, fb1, fb2, fb3]
            fsems = [sem_f0, sem_f1, sem_f2, sem_f3]
            NBUF = 4

            for j in range(NBUF):
                @pl.when(j < nfull)
                def _(j=j):
                    start_chunk(j, fbufs[j], fsems[j])

            @pl.when(head_needed)
            def _():
                base = pl.multiple_of(hbase, 8)
                pltpu.make_async_copy(
                    x_hbm.at[pl.ds(base, 8)], hbuf, sem_h).start()

            @pl.when(tail_needed)
            def _():
                base = pl.multiple_of(tbase, 8)
                pltpu.make_async_copy(
                    x_hbm.at[pl.ds(base, CR)], tbuf, sem_t).start()

            ninf16 = jnp.full((L,), _NEG_INF, jnp.float32)
            zero16 = jnp.zeros((L,), jnp.float32)
            for c in range(n_sub):
                omax_v[s_idx, pl.ds(c * L, L)] = ninf16
                osum_v[s_idx, pl.ds(c * L, L)] = zero16

            def wait(buf, sem, rows):
                pltpu.make_async_copy(
                    x_hbm.at[pl.ds(0, rows)], buf, sem).wait()

            nquads = lax.div(nfull + (NBUF - 1), NBUF)

            def quad(i, _):
                k0 = NBUF * i
                for j in range(NBUF):
                    @pl.when(k0 + j < nfull)
                    def _(j=j):
                        wait(fbufs[j], fsems[j], CR)
                        accum_rows(s_idx, fbufs[j], CR, lambda r: None)

                        @pl.when(k0 + j + NBUF < nfull)
                        def _():
                            start_chunk(k0 + j + NBUF, fbufs[j], fsems[j])

                return 0

            lax.fori_loop(0, nquads, quad, 0)

            @pl.when(head_needed)
            def _():
                wait(hbuf, sem_h, 8)
                accum_rows(
                    s_idx, hbuf, 8,
                    lambda r: jnp.logical_and(hbase + r >= lo,
                                              hbase + r < head_hi))

            @pl.when(tail_needed)
            def _():
                wait(tbuf, sem_t, CR)
                accum_rows(
                    s_idx, tbuf, CR,
                    lambda r: jnp.logical_and(tbase + r >= t_lo,
                                              tbase + r < hi))

            for c in range(n_sub):
                cs = pl.ds(c * L, L)
                m = omax_v[s_idx, cs]
                omax_v[s_idx, cs] = jnp.where(m == _NEG_INF, 0.0, m)
            return 0

        lax.fori_loop(0, seg_per_w, do_segment, 0)

        pltpu.sync_copy(omax_v, max_hbm.at[pl.ds(seg0, seg_per_w)])
        pltpu.sync_copy(osum_v, sum_hbm.at[pl.ds(seg0, seg_per_w)])

    return seg_reduce


def _merge_body(mx_ref, sm_ref, wa_ref, wb_ref, b_ref, o_ref):
    acc = lax.dot_general(mx_ref[...], wa_ref[...], (((1,), (1,)), ((), ())),
                          preferred_element_type=jnp.float32)
    acc = acc + lax.dot_general(sm_ref[...], wb_ref[...],
                                (((1,), (1,)), ((), ())),
                                preferred_element_type=jnp.float32)
    o_ref[...] = acc + b_ref[...]


def kernel(x, membership, W_merge, b_merge):
    n, d = x.shape
    bseg = NUM_SEGMENTS
    memb32 = membership.astype(jnp.int32)
    seg_max, seg_sum = _make_seg_reduce(n, d, bseg)(memb32, x)
    wa = W_merge[:, :d]
    wb = W_merge[:, d:]
    out = pl.pallas_call(
        _merge_body,
        out_shape=jax.ShapeDtypeStruct((bseg, d), jnp.float32),
    )(seg_max, seg_sum, wa, wb, b_merge.reshape(1, d))
    return out


# speedup vs baseline: 1.8350x; 1.8350x over previous
import functools

import jax
import jax.numpy as jnp
from jax import lax
from jax.experimental import pallas as pl
from jax.experimental.pallas import tpu as pltpu
from jax.experimental.pallas import tpu_sc as plsc

NUM_SEGMENTS = 512
NC = 2
NS = 16
L = 16
CR = 32

_NEG_INF = float("-inf")


@functools.lru_cache(maxsize=None)
def _make_seg_reduce(n, d, b):
    nw = NC * NS
    seg_per_w = b // nw
    n_sub = d // L
    assert n % 8 == 0 and n >= CR and d % L == 0 and b % nw == 0
    mesh = plsc.VectorSubcoreMesh(core_axis_name="c", subcore_axis_name="s",
                                  num_cores=NC, num_subcores=NS)

    @functools.partial(
        pl.kernel,
        out_type=(
            jax.ShapeDtypeStruct((b, d), jnp.float32),
            jax.ShapeDtypeStruct((b, d), jnp.float32),
        ),
        mesh=mesh,
        scratch_types=[
            pltpu.VMEM((n,), jnp.int32),
            pltpu.VMEM((8, d), jnp.float32),
            pltpu.VMEM((CR, d), jnp.float32),
            pltpu.VMEM((CR, d), jnp.float32),
            pltpu.VMEM((CR, d), jnp.float32),
            pltpu.VMEM((seg_per_w, d), jnp.float32),
            pltpu.VMEM((seg_per_w, d), jnp.float32),
            pltpu.SemaphoreType.DMA,
            pltpu.SemaphoreType.DMA,
            pltpu.SemaphoreType.DMA,
            pltpu.SemaphoreType.DMA,
        ],
        compiler_params=pltpu.CompilerParams(needs_layout_passes=False),
    )
    def seg_reduce(memb_hbm, x_hbm, max_hbm, sum_hbm,
                   memb_v, hbuf, fbuf0, fbuf1, tbuf, omax_v, osum_v,
                   sem_h, sem_f0, sem_f1, sem_t):
        wid = lax.axis_index("c") * NS + lax.axis_index("s")
        seg0 = wid * seg_per_w

        pltpu.sync_copy(memb_hbm, memb_v)

        targets = seg0 + lax.iota(jnp.int32, L)

        def lower_bound(tv):
            def step(_, carry):
                lo, hi = carry
                mid = lax.div(lo + hi, 2)
                vals = plsc.load_gather(memb_v, [jnp.minimum(mid, n - 1)])
                pred = vals < tv
                return jnp.where(pred, mid + 1, lo), jnp.where(pred, hi, mid)
            lo = jnp.zeros((L,), jnp.int32)
            hi = jnp.full((L,), n, jnp.int32)
            lo, hi = lax.fori_loop(0, 17, step, (lo, hi))
            return jnp.minimum(lo, n)

        starts = lower_bound(targets)
        ends = lower_bound(targets + 1)
        lanes = lax.iota(jnp.int32, L)

        def lane_extract(vec, idx):
            return jnp.sum(jnp.where(lanes == idx, vec, 0), axis=0)

        def accum_rows(s_idx, buf, nrows, valid_of_row):
            npart = 4 if nrows >= 8 else 2

            def cbody(c, _):
                cs = pl.ds(pl.multiple_of(c * L, L), L)
                pmax = [jnp.full((L,), _NEG_INF, jnp.float32)
                        for _ in range(npart)]
                psum = [jnp.zeros((L,), jnp.float32) for _ in range(npart)]
                for r in range(nrows):
                    v = buf[r, cs]
                    vmask = valid_of_row(r)
                    if vmask is None:
                        vm = vs = v
                    else:
                        vm = jnp.where(vmask, v, _NEG_INF)
                        vs = jnp.where(vmask, v, 0.0)
                    p = r % npart
                    pmax[p] = jnp.maximum(pmax[p], vm)
                    psum[p] = psum[p] + vs
                for p in range(1, npart):
                    pmax[0] = jnp.maximum(pmax[0], pmax[p])
                    psum[0] = psum[0] + psum[p]
                omax_v[s_idx, cs] = jnp.maximum(omax_v[s_idx, cs], pmax[0])
                osum_v[s_idx, cs] = osum_v[s_idx, cs] + psum[0]
                return 0

            lax.fori_loop(0, n_sub, cbody, 0)

        def do_segment(s_idx, _):
            lo = lane_extract(starts, s_idx)
            hi = lane_extract(ends, s_idx)
            lo8u = lax.div(lo + 7, 8) * 8
            nfull = lax.div(jnp.maximum(hi - lo8u, 0), CR)
            head_hi = jnp.minimum(lo8u, hi)
            head_needed = lo < head_hi
            hbase = jnp.minimum(jnp.maximum(lo8u - 8, 0), n - 8)
            t_lo = lo8u + nfull * CR
            tail_needed = t_lo < hi
            tbase = jnp.minimum(t_lo, n - CR)

            def start_chunk(k, buf, sem):
                base = pl.multiple_of(lo8u + k * CR, 8)
                pltpu.make_async_copy(
                    x_hbm.at[pl.ds(base, CR)], buf, sem).start()

            @pl.when(nfull > 0)
            def _():
                start_chunk(0, fbuf0, sem_f0)

            @pl.when(head_needed)
            def _():
                base = pl.multiple_of(hbase, 8)
                pltpu.make_async_copy(
                    x_hbm.at[pl.ds(base, 8)], hbuf, sem_h).start()

            @pl.when(tail_needed)
            def _():
                base = pl.multiple_of(tbase, 8)
                pltpu.make_async_copy(
                    x_hbm.at[pl.ds(base, CR)], tbuf, sem_t).start()

            ninf16 = jnp.full((L,), _NEG_INF, jnp.float32)
            zero16 = jnp.zeros((L,), jnp.float32)

            def init_c(c, _):
                cs = pl.ds(pl.multiple_of(c * L, L), L)
                omax_v[s_idx, cs] = ninf16
                osum_v[s_idx, cs] = zero16
                return 0

            lax.fori_loop(0, n_sub, init_c, 0)

            def wait(buf, sem, rows):
                pltpu.make_async_copy(
                    x_hbm.at[pl.ds(0, rows)], buf, sem).wait()

            npairs = lax.div(nfull + 1, 2)

            def pair(i, _):
                k0 = 2 * i
                wait(fbuf0, sem_f0, CR)

                @pl.when(k0 + 1 < nfull)
                def _():
                    start_chunk(k0 + 1, fbuf1, sem_f1)

                accum_rows(s_idx, fbuf0, CR, lambda r: None)

                @pl.when(k0 + 1 < nfull)
                def _():
                    wait(fbuf1, sem_f1, CR)

                    @pl.when(k0 + 2 < nfull)
                    def _():
                        start_chunk(k0 + 2, fbuf0, sem_f0)

                    accum_rows(s_idx, fbuf1, CR, lambda r: None)

                return 0

            lax.fori_loop(0, npairs, pair, 0)

            @pl.when(head_needed)
            def _():
                wait(hbuf, sem_h, 8)
                accum_rows(
                    s_idx, hbuf, 8,
                    lambda r: jnp.logical_and(hbase + r >= lo,
                                              hbase + r < head_hi))

            @pl.when(tail_needed)
            def _():
                wait(tbuf, sem_t, CR)
                accum_rows(
                    s_idx, tbuf, CR,
                    lambda r: jnp.logical_and(tbase + r >= t_lo,
                                              tbase + r < hi))

            def fin_c(c, _):
                cs = pl.ds(pl.multiple_of(c * L, L), L)
                m = omax_v[s_idx, cs]
                omax_v[s_idx, cs] = jnp.where(m == _NEG_INF, 0.0, m)
                return 0

            lax.fori_loop(0, n_sub, fin_c, 0)
            return 0

        lax.fori_loop(0, seg_per_w, do_segment, 0)

        pltpu.sync_copy(omax_v, max_hbm.at[pl.ds(seg0, seg_per_w)])
        pltpu.sync_copy(osum_v, sum_hbm.at[pl.ds(seg0, seg_per_w)])

    return seg_reduce


def _merge_body(mx_ref, sm_ref, wa_ref, wb_ref, b_ref, o_ref):
    acc = lax.dot_general(mx_ref[...], wa_ref[...], (((1,), (1,)), ((), ())),
                          preferred_element_type=jnp.float32)
    acc = acc + lax.dot_general(sm_ref[...], wb_ref[...],
                                (((1,), (1,)), ((), ())),
                                preferred_element_type=jnp.float32)
    o_ref[...] = acc + b_ref[...]


def kernel(x, membership, W_merge, b_merge):
    n, d = x.shape
    bseg = NUM_SEGMENTS
    memb32 = membership.astype(jnp.int32)
    seg_max, seg_sum = _make_seg_reduce(n, d, bseg)(memb32, x)
    wa = W_merge[:, :d]
    wb = W_merge[:, d:]
    out = pl.pallas_call(
        _merge_body,
        out_shape=jax.ShapeDtypeStruct((bseg, d), jnp.float32),
    )(seg_max, seg_sum, wa, wb, b_merge.reshape(1, d))
    return out


# speedup vs baseline: 1.9356x; 1.0548x over previous
import functools

import jax
import jax.numpy as jnp
from jax import lax
from jax.experimental import pallas as pl
from jax.experimental.pallas import tpu as pltpu
from jax.experimental.pallas import tpu_sc as plsc

NUM_SEGMENTS = 512
NC = 2
NS = 16
L = 16
CR = 32

_NEG_INF = float("-inf")


@functools.lru_cache(maxsize=None)
def _make_seg_reduce(n, d, b):
    nw = NC * NS
    seg_per_w = b // nw
    n_sub = d // L
    assert n % 8 == 0 and n >= CR and d % L == 0 and b % nw == 0
    mesh = plsc.VectorSubcoreMesh(core_axis_name="c", subcore_axis_name="s",
                                  num_cores=NC, num_subcores=NS)

    @functools.partial(
        pl.kernel,
        out_type=(
            jax.ShapeDtypeStruct((b, d), jnp.float32),
            jax.ShapeDtypeStruct((b, d), jnp.float32),
        ),
        mesh=mesh,
        scratch_types=[
            pltpu.VMEM((n,), jnp.int32),
            pltpu.VMEM((8, d), jnp.float32),
            pltpu.VMEM((CR, d), jnp.float32),
            pltpu.VMEM((CR, d), jnp.float32),
            pltpu.VMEM((CR, d), jnp.float32),
            pltpu.VMEM((CR, d), jnp.float32),
            pltpu.VMEM((CR, d), jnp.float32),
            pltpu.VMEM((seg_per_w, d), jnp.float32),
            pltpu.VMEM((seg_per_w, d), jnp.float32),
            pltpu.SemaphoreType.DMA,
            pltpu.SemaphoreType.DMA,
            pltpu.SemaphoreType.DMA,
            pltpu.SemaphoreType.DMA,
            pltpu.SemaphoreType.DMA,
            pltpu.SemaphoreType.DMA,
        ],
        compiler_params=pltpu.CompilerParams(needs_layout_passes=False),
    )
    def seg_reduce(memb_hbm, x_hbm, max_hbm, sum_hbm,
                   memb_v, hbuf, fb0, fb1, fb2, fb3, tbuf, omax_v, osum_v,
                   sem_h, sem_f0, sem_f1, sem_f2, sem_f3, sem_t):
        wid = lax.axis_index("c") * NS + lax.axis_index("s")
        seg0 = wid * seg_per_w

        pltpu.sync_copy(memb_hbm, memb_v)

        targets = seg0 + lax.iota(jnp.int32, L)

        def lower_bound(tv):
            def step(_, carry):
                lo, hi = carry
                mid = lax.div(lo + hi, 2)
                vals = plsc.load_gather(memb_v, [jnp.minimum(mid, n - 1)])
                pred = vals < tv
                return jnp.where(pred, mid + 1, lo), jnp.where(pred, hi, mid)
            lo = jnp.zeros((L,), jnp.int32)
            hi = jnp.full((L,), n, jnp.int32)
            lo, hi = lax.fori_loop(0, 17, step, (lo, hi))
            return jnp.minimum(lo, n)

        starts = lower_bound(targets)
        ends = lower_bound(targets + 1)
        lanes = lax.iota(jnp.int32, L)

        def lane_extract(vec, idx):
            return jnp.sum(jnp.where(lanes == idx, vec, 0), axis=0)

        def accum_rows(s_idx, buf, nrows, valid_of_row):
            npart = 4 if nrows >= 8 else 2

            def cbody(c, _):
                cs = pl.ds(pl.multiple_of(c * L, L), L)
                pmax = [jnp.full((L,), _NEG_INF, jnp.float32)
                        for _ in range(npart)]
                psum = [jnp.zeros((L,), jnp.float32) for _ in range(npart)]
                for r in range(nrows):
                    v = buf[r, cs]
                    vmask = valid_of_row(r)
                    if vmask is None:
                        vm = vs = v
                    else:
                        vm = jnp.where(vmask, v, _NEG_INF)
                        vs = jnp.where(vmask, v, 0.0)
                    p = r % npart
                    pmax[p] = jnp.maximum(pmax[p], vm)
                    psum[p] = psum[p] + vs
                for p in range(1, npart):
                    pmax[0] = jnp.maximum(pmax[0], pmax[p])
                    psum[0] = psum[0] + psum[p]
                omax_v[s_idx, cs] = jnp.maximum(omax_v[s_idx, cs], pmax[0])
                osum_v[s_idx, cs] = osum_v[s_idx, cs] + psum[0]
                return 0

            lax.fori_loop(0, n_sub, cbody, 0)

        def do_segment(s_idx, _):
            lo = lane_extract(starts, s_idx)
            hi = lane_extract(ends, s_idx)
            lo8u = lax.div(lo + 7, 8) * 8
            nfull = lax.div(jnp.maximum(hi - lo8u, 0), CR)
            head_hi = jnp.minimum(lo8u, hi)
            head_needed = lo < head_hi
            hbase = jnp.minimum(jnp.maximum(lo8u - 8, 0), n - 8)
            t_lo = lo8u + nfull * CR
            tail_needed = t_lo < hi
            tbase = jnp.minimum(t_lo, n - CR)

            def start_chunk(k, buf, sem):
                base = pl.multiple_of(lo8u + k * CR, 8)
                pltpu.make_async_copy(
                    x_hbm.at[pl.ds(base, CR)], buf, sem).start()

            fbufs = [fb0, fb1, fb2, fb3]
            fsems = [sem_f0, sem_f1, sem_f2, sem_f3]
            NBUF = 4

            for j in range(NBUF):
                @pl.when(j < nfull)
                def _(j=j):
                    start_chunk(j, fbufs[j], fsems[j])

            @pl.when(head_needed)
            def _():
                base = pl.multiple_of(hbase, 8)
                pltpu.make_async_copy(
                    x_hbm.at[pl.ds(base, 8)], hbuf, sem_h).start()

            @pl.when(tail_needed)
            def _():
                base = pl.multiple_of(tbase, 8)
                pltpu.make_async_copy(
                    x_hbm.at[pl.ds(base, CR)], tbuf, sem_t).start()

            ninf16 = jnp.full((L,), _NEG_INF, jnp.float32)
            zero16 = jnp.zeros((L,), jnp.float32)

            def init_c(c, _):
                cs = pl.ds(pl.multiple_of(c * L, L), L)
                omax_v[s_idx, cs] = ninf16
                osum_v[s_idx, cs] = zero16
                return 0

            lax.fori_loop(0, n_sub, init_c, 0)

            def wait(buf, sem, rows):
                pltpu.make_async_copy(
                    x_hbm.at[pl.ds(0, rows)], buf, sem).wait()

            nquads = lax.div(nfull + (NBUF - 1), NBUF)

            def quad(i, _):
                k0 = NBUF * i
                for j in range(NBUF):
                    @pl.when(k0 + j < nfull)
                    def _(j=j):
                        wait(fbufs[j], fsems[j], CR)
                        accum_rows(s_idx, fbufs[j], CR, lambda r: None)

                        @pl.when(k0 + j + NBUF < nfull)
                        def _():
                            start_chunk(k0 + j + NBUF, fbufs[j], fsems[j])

                return 0

            lax.fori_loop(0, nquads, quad, 0)

            @pl.when(head_needed)
            def _():
                wait(hbuf, sem_h, 8)
                accum_rows(
                    s_idx, hbuf, 8,
                    lambda r: jnp.logical_and(hbase + r >= lo,
                                              hbase + r < head_hi))

            @pl.when(tail_needed)
            def _():
                wait(tbuf, sem_t, CR)
                accum_rows(
                    s_idx, tbuf, CR,
                    lambda r: jnp.logical_and(tbase + r >= t_lo,
                                              tbase + r < hi))

            def fin_c(c, _):
                cs = pl.ds(pl.multiple_of(c * L, L), L)
                m = omax_v[s_idx, cs]
                omax_v[s_idx, cs] = jnp.where(m == _NEG_INF, 0.0, m)
                return 0

            lax.fori_loop(0, n_sub, fin_c, 0)
            return 0

        lax.fori_loop(0, seg_per_w, do_segment, 0)

        pltpu.sync_copy(omax_v, max_hbm.at[pl.ds(seg0, seg_per_w)])
        pltpu.sync_copy(osum_v, sum_hbm.at[pl.ds(seg0, seg_per_w)])

    return seg_reduce


def _merge_body(mx_ref, sm_ref, wa_ref, wb_ref, b_ref, o_ref):
    acc = lax.dot_general(mx_ref[...], wa_ref[...], (((1,), (1,)), ((), ())),
                          preferred_element_type=jnp.float32)
    acc = acc + lax.dot_general(sm_ref[...], wb_ref[...],
                                (((1,), (1,)), ((), ())),
                                preferred_element_type=jnp.float32)
    o_ref[...] = acc + b_ref[...]


def kernel(x, membership, W_merge, b_merge):
    n, d = x.shape
    bseg = NUM_SEGMENTS
    memb32 = membership.astype(jnp.int32)
    seg_max, seg_sum = _make_seg_reduce(n, d, bseg)(memb32, x)
    wa = W_merge[:, :d]
    wb = W_merge[:, d:]
    out = pl.pallas_call(
        _merge_body,
        out_shape=jax.ShapeDtypeStruct((bseg, d), jnp.float32),
    )(seg_max, seg_sum, wa, wb, b_merge.reshape(1, d))
    return out


# speedup vs baseline: 2.0219x; 1.0446x over previous
import functools

import jax
import jax.numpy as jnp
from jax import lax
from jax.experimental import pallas as pl
from jax.experimental.pallas import tpu as pltpu
from jax.experimental.pallas import tpu_sc as plsc

NUM_SEGMENTS = 512
NC = 2
NS = 16
L = 16
CR = 32

_NEG_INF = float("-inf")


@functools.lru_cache(maxsize=None)
def _make_seg_reduce(n, d, b):
    nw = NC * NS
    seg_per_w = b // nw
    n_sub = d // L
    assert n % 8 == 0 and n >= CR and d % L == 0 and b % nw == 0
    mesh = plsc.VectorSubcoreMesh(core_axis_name="c", subcore_axis_name="s",
                                  num_cores=NC, num_subcores=NS)

    @functools.partial(
        pl.kernel,
        out_type=(
            jax.ShapeDtypeStruct((b, d), jnp.float32),
            jax.ShapeDtypeStruct((b, d), jnp.float32),
        ),
        mesh=mesh,
        scratch_types=[
            pltpu.VMEM((n,), jnp.int32),
            pltpu.VMEM((8, d), jnp.float32),
            pltpu.VMEM((CR, d), jnp.float32),
            pltpu.VMEM((CR, d), jnp.float32),
            pltpu.VMEM((CR, d), jnp.float32),
            pltpu.VMEM((CR, d), jnp.float32),
            pltpu.VMEM((CR, d), jnp.float32),
            pltpu.VMEM((seg_per_w, d), jnp.float32),
            pltpu.VMEM((seg_per_w, d), jnp.float32),
            pltpu.SemaphoreType.DMA,
            pltpu.SemaphoreType.DMA,
            pltpu.SemaphoreType.DMA,
            pltpu.SemaphoreType.DMA,
            pltpu.SemaphoreType.DMA,
            pltpu.SemaphoreType.DMA,
        ],
        compiler_params=pltpu.CompilerParams(needs_layout_passes=False),
    )
    def seg_reduce(memb_hbm, x_hbm, max_hbm, sum_hbm,
                   memb_v, hbuf, fb0, fb1, fb2, fb3, tbuf, omax_v, osum_v,
                   sem_h, sem_f0, sem_f1, sem_f2, sem_f3, sem_t):
        wid = lax.axis_index("c") * NS + lax.axis_index("s")
        seg0 = wid * seg_per_w

        pltpu.sync_copy(memb_hbm, memb_v)

        targets = seg0 + lax.iota(jnp.int32, L)

        def lower_bound(tv):
            def step(_, carry):
                lo, hi = carry
                mid = lax.div(lo + hi, 2)
                vals = plsc.load_gather(memb_v, [jnp.minimum(mid, n - 1)])
                pred = vals < tv
                return jnp.where(pred, mid + 1, lo), jnp.where(pred, hi, mid)
            lo = jnp.zeros((L,), jnp.int32)
            hi = jnp.full((L,), n, jnp.int32)
            lo, hi = lax.fori_loop(0, 17, step, (lo, hi))
            return jnp.minimum(lo, n)

        starts = lower_bound(targets)
        ends = lower_bound(targets + 1)
        lanes = lax.iota(jnp.int32, L)

        def lane_extract(vec, idx):
            return jnp.sum(jnp.where(lanes == idx, vec, 0), axis=0)

        def accum_rows(s_idx, buf, nrows, valid_of_row, row_offset=0):
            npart = 4 if nrows >= 8 else 2

            def cbody(c, _):
                cs = pl.ds(pl.multiple_of(c * L, L), L)
                pmax = [jnp.full((L,), _NEG_INF, jnp.float32)
                        for _ in range(npart)]
                psum = [jnp.zeros((L,), jnp.float32) for _ in range(npart)]
                for r in range(nrows):
                    v = buf[row_offset + r, cs]
                    vmask = valid_of_row(r)
                    if vmask is None:
                        vm = vs = v
                    else:
                        vm = jnp.where(vmask, v, _NEG_INF)
                        vs = jnp.where(vmask, v, 0.0)
                    p = r % npart
                    pmax[p] = jnp.maximum(pmax[p], vm)
                    psum[p] = psum[p] + vs
                for p in range(1, npart):
                    pmax[0] = jnp.maximum(pmax[0], pmax[p])
                    psum[0] = psum[0] + psum[p]
                omax_v[s_idx, cs] = jnp.maximum(omax_v[s_idx, cs], pmax[0])
                osum_v[s_idx, cs] = osum_v[s_idx, cs] + psum[0]
                return 0

            lax.fori_loop(0, n_sub, cbody, 0)

        def do_segment(s_idx, _):
            lo = lane_extract(starts, s_idx)
            hi = lane_extract(ends, s_idx)
            lo8u = lax.div(lo + 7, 8) * 8
            nfull = lax.div(jnp.maximum(hi - lo8u, 0), CR)
            head_hi = jnp.minimum(lo8u, hi)
            head_needed = lo < head_hi
            hbase = jnp.minimum(jnp.maximum(lo8u - 8, 0), n - 8)
            t_lo = lo8u + nfull * CR
            nt8 = lax.div(jnp.maximum(hi - t_lo, 0), 8)
            t8s = t_lo + nt8 * 8
            t8e = jnp.minimum(t8s, n - 8)

            def start_chunk(k, buf, sem):
                base = pl.multiple_of(lo8u + k * CR, 8)
                pltpu.make_async_copy(
                    x_hbm.at[pl.ds(base, CR)], buf, sem).start()

            fbufs = [fb0, fb1, fb2, fb3]
            fsems = [sem_f0, sem_f1, sem_f2, sem_f3]
            NBUF = 4

            for j in range(NBUF):
                @pl.when(j < nfull)
                def _(j=j):
                    start_chunk(j, fbufs[j], fsems[j])

            @pl.when(head_needed)
            def _():
                base = pl.multiple_of(hbase, 8)
                pltpu.make_async_copy(
                    x_hbm.at[pl.ds(base, 8)], hbuf, sem_h).start()

            for j in range(CR // 8 - 1):
                @pl.when(j < nt8)
                def _(j=j):
                    base = pl.multiple_of(t_lo + j * 8, 8)
                    pltpu.make_async_copy(
                        x_hbm.at[pl.ds(base, 8)],
                        tbuf.at[pl.ds(j * 8, 8)], sem_t).start()

            @pl.when(t8s < hi)
            def _():
                base = pl.multiple_of(t8e, 8)
                pltpu.make_async_copy(
                    x_hbm.at[pl.ds(base, 8)],
                    tbuf.at[pl.ds(CR - 8, 8)], sem_t).start()

            ninf16 = jnp.full((L,), _NEG_INF, jnp.float32)
            zero16 = jnp.zeros((L,), jnp.float32)

            def init_c(c, _):
                cs = pl.ds(pl.multiple_of(c * L, L), L)
                omax_v[s_idx, cs] = ninf16
                osum_v[s_idx, cs] = zero16
                return 0

            lax.fori_loop(0, n_sub, init_c, 0)

            def wait(buf, sem, rows):
                pltpu.make_async_copy(
                    x_hbm.at[pl.ds(0, rows)], buf, sem).wait()

            nquads = lax.div(nfull + (NBUF - 1), NBUF)

            def quad(i, _):
                k0 = NBUF * i
                for j in range(NBUF):
                    @pl.when(k0 + j < nfull)
                    def _(j=j):
                        wait(fbufs[j], fsems[j], CR)
                        accum_rows(s_idx, fbufs[j], CR, lambda r: None)

                        @pl.when(k0 + j + NBUF < nfull)
                        def _():
                            start_chunk(k0 + j + NBUF, fbufs[j], fsems[j])

                return 0

            lax.fori_loop(0, nquads, quad, 0)

            @pl.when(head_needed)
            def _():
                wait(hbuf, sem_h, 8)
                accum_rows(
                    s_idx, hbuf, 8,
                    lambda r: jnp.logical_and(hbase + r >= lo,
                                              hbase + r < head_hi))

            for j in range(CR // 8 - 1):
                @pl.when(j < nt8)
                def _(j=j):
                    pltpu.make_async_copy(
                        x_hbm.at[pl.ds(0, 8)],
                        tbuf.at[pl.ds(j * 8, 8)], sem_t).wait()
                    accum_rows(s_idx, tbuf, 8, lambda r: None,
                               row_offset=j * 8)

            @pl.when(t8s < hi)
            def _():
                pltpu.make_async_copy(
                    x_hbm.at[pl.ds(0, 8)],
                    tbuf.at[pl.ds(CR - 8, 8)], sem_t).wait()
                accum_rows(
                    s_idx, tbuf, 8,
                    lambda r: jnp.logical_and(t8e + r >= t8s,
                                              t8e + r < hi),
                    row_offset=CR - 8)

            def fin_c(c, _):
                cs = pl.ds(pl.multiple_of(c * L, L), L)
                m = omax_v[s_idx, cs]
                omax_v[s_idx, cs] = jnp.where(m == _NEG_INF, 0.0, m)
                return 0

            lax.fori_loop(0, n_sub, fin_c, 0)
            return 0

        lax.fori_loop(0, seg_per_w, do_segment, 0)

        pltpu.sync_copy(omax_v, max_hbm.at[pl.ds(seg0, seg_per_w)])
        pltpu.sync_copy(osum_v, sum_hbm.at[pl.ds(seg0, seg_per_w)])

    return seg_reduce


def _merge_body(mx_ref, sm_ref, wa_ref, wb_ref, b_ref, o_ref):
    acc = lax.dot_general(mx_ref[...], wa_ref[...], (((1,), (1,)), ((), ())),
                          preferred_element_type=jnp.float32)
    acc = acc + lax.dot_general(sm_ref[...], wb_ref[...],
                                (((1,), (1,)), ((), ())),
                                preferred_element_type=jnp.float32)
    o_ref[...] = acc + b_ref[...]


def kernel(x, membership, W_merge, b_merge):
    n, d = x.shape
    bseg = NUM_SEGMENTS
    memb32 = membership.astype(jnp.int32)
    seg_max, seg_sum = _make_seg_reduce(n, d, bseg)(memb32, x)
    wa = W_merge[:, :d]
    wb = W_merge[:, d:]
    out = pl.pallas_call(
        _merge_body,
        out_shape=jax.ShapeDtypeStruct((bseg, d), jnp.float32),
    )(seg_max, seg_sum, wa, wb, b_merge.reshape(1, d))
    return out


# speedup vs baseline: 2.1364x; 1.0566x over previous
import functools

import jax
import jax.numpy as jnp
from jax import lax
from jax.experimental import pallas as pl
from jax.experimental.pallas import tpu as pltpu
from jax.experimental.pallas import tpu_sc as plsc

NUM_SEGMENTS = 512
NC = 2
NS = 16
L = 16
CR = 32

_NEG_INF = float("-inf")


@functools.lru_cache(maxsize=None)
def _make_seg_reduce(n, d, b):
    nw = NC * NS
    seg_per_w = b // nw
    n_sub = d // L
    assert n % 8 == 0 and n >= CR and d % L == 0 and b % nw == 0
    mesh = plsc.VectorSubcoreMesh(core_axis_name="c", subcore_axis_name="s",
                                  num_cores=NC, num_subcores=NS)

    @functools.partial(
        pl.kernel,
        out_type=(
            jax.ShapeDtypeStruct((b, d), jnp.float32),
            jax.ShapeDtypeStruct((b, d), jnp.float32),
        ),
        mesh=mesh,
        scratch_types=[
            pltpu.VMEM((n,), jnp.int32),
            pltpu.VMEM((8, d), jnp.float32),
            pltpu.VMEM((CR, d), jnp.float32),
            pltpu.VMEM((CR, d), jnp.float32),
            pltpu.VMEM((CR, d), jnp.float32),
            pltpu.VMEM((CR, d), jnp.float32),
            pltpu.VMEM((CR, d), jnp.float32),
            pltpu.VMEM((CR, d), jnp.float32),
            pltpu.VMEM((CR, d), jnp.float32),
            pltpu.VMEM((seg_per_w, d), jnp.float32),
            pltpu.VMEM((seg_per_w, d), jnp.float32),
            pltpu.SemaphoreType.DMA,
            pltpu.SemaphoreType.DMA,
            pltpu.SemaphoreType.DMA,
            pltpu.SemaphoreType.DMA,
            pltpu.SemaphoreType.DMA,
            pltpu.SemaphoreType.DMA,
            pltpu.SemaphoreType.DMA,
            pltpu.SemaphoreType.DMA,
        ],
        compiler_params=pltpu.CompilerParams(needs_layout_passes=False),
    )
    def seg_reduce(memb_hbm, x_hbm, max_hbm, sum_hbm,
                   memb_v, hbuf, fb0, fb1, fb2, fb3, fb4, fb5, tbuf,
                   omax_v, osum_v,
                   sem_h, sem_f0, sem_f1, sem_f2, sem_f3, sem_f4, sem_f5,
                   sem_t):
        wid = lax.axis_index("c") * NS + lax.axis_index("s")
        seg0 = wid * seg_per_w

        pltpu.sync_copy(memb_hbm, memb_v)

        targets = seg0 + lax.iota(jnp.int32, L)

        def lower_bound(tv):
            def step(_, carry):
                lo, hi = carry
                mid = lax.div(lo + hi, 2)
                vals = plsc.load_gather(memb_v, [jnp.minimum(mid, n - 1)])
                pred = vals < tv
                return jnp.where(pred, mid + 1, lo), jnp.where(pred, hi, mid)
            lo = jnp.zeros((L,), jnp.int32)
            hi = jnp.full((L,), n, jnp.int32)
            lo, hi = lax.fori_loop(0, 17, step, (lo, hi))
            return jnp.minimum(lo, n)

        starts = lower_bound(targets)
        ends = lower_bound(targets + 1)
        lanes = lax.iota(jnp.int32, L)

        def lane_extract(vec, idx):
            return jnp.sum(jnp.where(lanes == idx, vec, 0), axis=0)

        def accum_rows(s_idx, buf, nrows, valid_of_row, row_offset=0):
            npart = 4 if nrows >= 8 else 2

            def cbody(c, _):
                cs = pl.ds(pl.multiple_of(c * L, L), L)
                pmax = [jnp.full((L,), _NEG_INF, jnp.float32)
                        for _ in range(npart)]
                psum = [jnp.zeros((L,), jnp.float32) for _ in range(npart)]
                for r in range(nrows):
                    v = buf[row_offset + r, cs]
                    vmask = valid_of_row(r)
                    if vmask is None:
                        vm = vs = v
                    else:
                        vm = jnp.where(vmask, v, _NEG_INF)
                        vs = jnp.where(vmask, v, 0.0)
                    p = r % npart
                    pmax[p] = jnp.maximum(pmax[p], vm)
                    psum[p] = psum[p] + vs
                for p in range(1, npart):
                    pmax[0] = jnp.maximum(pmax[0], pmax[p])
                    psum[0] = psum[0] + psum[p]
                omax_v[s_idx, cs] = jnp.maximum(omax_v[s_idx, cs], pmax[0])
                osum_v[s_idx, cs] = osum_v[s_idx, cs] + psum[0]
                return 0

            lax.fori_loop(0, n_sub, cbody, 0)

        def do_segment(s_idx, last_tb):
            lo = lane_extract(starts, s_idx)
            hi = lane_extract(ends, s_idx)
            lo8u = lax.div(lo + 7, 8) * 8
            nfull = lax.div(jnp.maximum(hi - lo8u, 0), CR)
            head_hi = jnp.minimum(lo8u, hi)
            head_needed = lo < head_hi
            hbase = jnp.minimum(jnp.maximum(lo8u - 8, 0), n - 8)
            t_lo = lo8u + nfull * CR
            nt8 = lax.div(jnp.maximum(hi - t_lo, 0), 8)
            t8s = t_lo + nt8 * 8
            t8e = jnp.minimum(t8s, n - 8)

            def start_chunk(k, buf, sem):
                base = pl.multiple_of(lo8u + k * CR, 8)
                pltpu.make_async_copy(
                    x_hbm.at[pl.ds(base, CR)], buf, sem).start()

            fbufs = [fb0, fb1, fb2, fb3, fb4, fb5]
            fsems = [sem_f0, sem_f1, sem_f2, sem_f3, sem_f4, sem_f5]
            NBUF = 6

            for j in range(NBUF):
                @pl.when(j < nfull)
                def _(j=j):
                    start_chunk(j, fbufs[j], fsems[j])

            reuse_head = jnp.logical_and(head_needed, last_tb == hbase)
            dma_head = jnp.logical_and(head_needed, last_tb != hbase)

            @pl.when(dma_head)
            def _():
                base = pl.multiple_of(hbase, 8)
                pltpu.make_async_copy(
                    x_hbm.at[pl.ds(base, 8)], hbuf, sem_h).start()

            ninf16 = jnp.full((L,), _NEG_INF, jnp.float32)
            zero16 = jnp.zeros((L,), jnp.float32)

            def init_c(c, _):
                cs = pl.ds(pl.multiple_of(c * L, L), L)
                omax_v[s_idx, cs] = ninf16
                osum_v[s_idx, cs] = zero16
                return 0

            lax.fori_loop(0, n_sub, init_c, 0)

            head_valid = lambda r: jnp.logical_and(hbase + r >= lo,
                                                   hbase + r < head_hi)

            @pl.when(dma_head)
            def _():
                pltpu.make_async_copy(
                    x_hbm.at[pl.ds(0, 8)], hbuf, sem_h).wait()
                accum_rows(s_idx, hbuf, 8, head_valid)

            @pl.when(reuse_head)
            def _():
                accum_rows(s_idx, tbuf, 8, head_valid, row_offset=CR - 8)

            for j in range(CR // 8 - 1):
                @pl.when(j < nt8)
                def _(j=j):
                    base = pl.multiple_of(t_lo + j * 8, 8)
                    pltpu.make_async_copy(
                        x_hbm.at[pl.ds(base, 8)],
                        tbuf.at[pl.ds(j * 8, 8)], sem_t).start()

            @pl.when(t8s < hi)
            def _():
                base = pl.multiple_of(t8e, 8)
                pltpu.make_async_copy(
                    x_hbm.at[pl.ds(base, 8)],
                    tbuf.at[pl.ds(CR - 8, 8)], sem_t).start()

            def wait(buf, sem, rows):
                pltpu.make_async_copy(
                    x_hbm.at[pl.ds(0, rows)], buf, sem).wait()

            nquads = lax.div(nfull + (NBUF - 1), NBUF)

            def quad(i, _):
                k0 = NBUF * i
                for j in range(NBUF):
                    @pl.when(k0 + j < nfull)
                    def _(j=j):
                        wait(fbufs[j], fsems[j], CR)
                        accum_rows(s_idx, fbufs[j], CR, lambda r: None)

                        @pl.when(k0 + j + NBUF < nfull)
                        def _():
                            start_chunk(k0 + j + NBUF, fbufs[j], fsems[j])

                return 0

            lax.fori_loop(0, nquads, quad, 0)

            for j in range(CR // 8 - 1):
                @pl.when(j < nt8)
                def _(j=j):
                    pltpu.make_async_copy(
                        x_hbm.at[pl.ds(0, 8)],
                        tbuf.at[pl.ds(j * 8, 8)], sem_t).wait()
                    accum_rows(s_idx, tbuf, 8, lambda r: None,
                               row_offset=j * 8)

            @pl.when(t8s < hi)
            def _():
                pltpu.make_async_copy(
                    x_hbm.at[pl.ds(0, 8)],
                    tbuf.at[pl.ds(CR - 8, 8)], sem_t).wait()
                accum_rows(
                    s_idx, tbuf, 8,
                    lambda r: jnp.logical_and(t8e + r >= t8s,
                                              t8e + r < hi),
                    row_offset=CR - 8)

            def fin_c(c, _):
                cs = pl.ds(pl.multiple_of(c * L, L), L)
                m = omax_v[s_idx, cs]
                omax_v[s_idx, cs] = jnp.where(m == _NEG_INF, 0.0, m)
                return 0

            lax.fori_loop(0, n_sub, fin_c, 0)
            return jnp.where(t8s < hi, t8e, last_tb)

        lax.fori_loop(0, seg_per_w, do_segment, jnp.int32(-1))

        pltpu.sync_copy(omax_v, max_hbm.at[pl.ds(seg0, seg_per_w)])
        pltpu.sync_copy(osum_v, sum_hbm.at[pl.ds(seg0, seg_per_w)])

    return seg_reduce


def _merge_body(mx_ref, sm_ref, wa_ref, wb_ref, b_ref, o_ref):
    acc = lax.dot_general(mx_ref[...], wa_ref[...], (((1,), (1,)), ((), ())),
                          preferred_element_type=jnp.float32)
    acc = acc + lax.dot_general(sm_ref[...], wb_ref[...],
                                (((1,), (1,)), ((), ())),
                                preferred_element_type=jnp.float32)
    o_ref[...] = acc + b_ref[...]


def kernel(x, membership, W_merge, b_merge):
    n, d = x.shape
    bseg = NUM_SEGMENTS
    memb32 = membership.astype(jnp.int32)
    seg_max, seg_sum = _make_seg_reduce(n, d, bseg)(memb32, x)
    wa = W_merge[:, :d]
    wb = W_merge[:, d:]
    out = pl.pallas_call(
        _merge_body,
        out_shape=jax.ShapeDtypeStruct((bseg, d), jnp.float32),
    )(seg_max, seg_sum, wa, wb, b_merge.reshape(1, d))
    return out
